# baseline (speedup 1.0000x reference)
import jax
import jax.numpy as jnp
from jax import lax
from jax.experimental import pallas as pl
from jax.experimental.pallas import tpu as pltpu
from jax.experimental.pallas import tpu_sc as plsc

_INFO = plsc.get_sparse_core_info()
_NC, _NS, _L = _INFO.num_cores, _INFO.num_subcores, _INFO.num_lanes
_NW = _NC * _NS

_B, _C, _T_IN = 8, 128, 48000
_STRIDE = 3
_T_OUT = _T_IN // _STRIDE

_IN_TOTAL = _B * _C * _T_IN
_OUT_TOTAL = _IN_TOTAL // _STRIDE

_IN_PER_W = _IN_TOTAL // _NW
_OUT_PER_W = _OUT_TOTAL // _NW

_CHUNK_IN = 48000
_CHUNK_OUT = _CHUNK_IN // _STRIDE
_STEPS = _IN_PER_W // _CHUNK_IN
_UNROLL = 8
_GATHER_ITERS = _CHUNK_OUT // (_L * _UNROLL)


def _compact_chunk(in_buf, out_buf):
  lane = lax.iota(jnp.int32, _L)
  idx0 = _STRIDE * lane

  def body(i, idx):
    base = i * (_L * _UNROLL)
    for u in range(_UNROLL):
      vals = plsc.load_gather(in_buf, [idx + (_STRIDE * _L) * u])
      out_buf[pl.ds(base + u * _L, _L)] = vals
    return idx + (_STRIDE * _L * _UNROLL)

  lax.fori_loop(0, _GATHER_ITERS, body, idx0, unroll=False)


def _sc_body(x_hbm, out_hbm, in_buf0, in_buf1, out_buf0, out_buf1,
             in_sems, out_sems):
  wid = lax.axis_index("s") * _NC + lax.axis_index("c")
  in_base = wid * _IN_PER_W
  out_base = wid * _OUT_PER_W
  in_bufs = (in_buf0, in_buf1)
  out_bufs = (out_buf0, out_buf1)

  def start_in(g):
    slot = g % 2
    return pltpu.async_copy(
        x_hbm.at[pl.ds(in_base + g * _CHUNK_IN, _CHUNK_IN)],
        in_bufs[slot],
        in_sems.at[slot],
    )

  def start_out(g):
    slot = g % 2
    return pltpu.async_copy(
        out_bufs[slot],
        out_hbm.at[pl.ds(out_base + g * _CHUNK_OUT, _CHUNK_OUT)],
        out_sems.at[slot],
    )

  in_copies = {0: start_in(0), 1: start_in(1)}
  out_copies = {}
  for g in range(_STEPS):
    slot = g % 2
    in_copies.pop(g).wait()
    if g >= 2:
      out_copies.pop(g - 2).wait()
    _compact_chunk(in_bufs[slot], out_bufs[slot])
    out_copies[g] = start_out(g)
    if g + 2 < _STEPS:
      in_copies[g + 2] = start_in(g + 2)
  out_copies.pop(_STEPS - 2).wait()
  out_copies.pop(_STEPS - 1).wait()


@jax.jit
def kernel(x):
  x_flat = x.reshape(-1)
  mesh = plsc.VectorSubcoreMesh(core_axis_name="c", subcore_axis_name="s")
  out_flat = pl.kernel(
      _sc_body,
      out_type=jax.ShapeDtypeStruct((_OUT_TOTAL,), jnp.float32),
      mesh=mesh,
      scratch_types=[
          pltpu.VMEM((_CHUNK_IN,), jnp.float32),
          pltpu.VMEM((_CHUNK_IN,), jnp.float32),
          pltpu.VMEM((_CHUNK_OUT,), jnp.float32),
          pltpu.VMEM((_CHUNK_OUT,), jnp.float32),
          pltpu.SemaphoreType.DMA((2,)),
          pltpu.SemaphoreType.DMA((2,)),
      ],
      compiler_params=pltpu.CompilerParams(needs_layout_passes=False),
      name="resample_nearest_sc",
  )(x_flat)
  return out_flat.reshape(_B, _C, _T_OUT)


# speedup vs baseline: 3.2668x; 3.2668x over previous
import jax
import jax.numpy as jnp
from jax import lax
from jax.experimental import pallas as pl
from jax.experimental.pallas import tpu as pltpu
from jax.experimental.pallas import tpu_sc as plsc

_INFO = plsc.get_sparse_core_info()
_NC, _NS, _L = _INFO.num_cores, _INFO.num_subcores, _INFO.num_lanes
_NW = _NC * _NS

_B, _C, _T_IN = 8, 128, 48000
_STRIDE = 3
_T_OUT = _T_IN // _STRIDE
_SUB = 8
_LANES = 128
_TILE = _SUB * _LANES
_CB = _C // _SUB
_JI = _T_IN // _LANES
_JO = _T_OUT // _LANES

_IN_TOTAL = _B * _C * _T_IN
_OUT_TOTAL = _IN_TOTAL // _STRIDE
_IN_PER_W = _IN_TOTAL // _NW
_OUT_PER_W = _OUT_TOTAL // _NW

_WO = 5
_WI = _WO * _STRIDE
_CHUNK_IN = _WI * _TILE
_CHUNK_OUT = _WO * _TILE
_STEPS = _IN_PER_W // _CHUNK_IN
_GROUPS = _LANES // _L


def _compact_chunk(in_buf, out_buf, idx_by_g):
  @plsc.parallel_loop(0, _WO * _SUB, unroll=4)
  def _(i):
    ot = i >> 3
    off_in = (i << 7) + (ot << 11)
    off_out = i * _LANES
    for g in range(_GROUPS):
      vals = plsc.load_gather(in_buf, [idx_by_g[g] + off_in])
      out_buf[pl.ds(off_out + g * _L, _L)] = vals


def _sc_body(x_hbm, out_hbm, in_buf0, in_buf1, in_buf2, in_buf3,
             out_buf0, out_buf1, in_sems, out_sems):
  wid = lax.axis_index("s") * _NC + lax.axis_index("c")
  in_base = wid * _IN_PER_W
  out_base = wid * _OUT_PER_W
  in_bufs = (in_buf0, in_buf1, in_buf2, in_buf3)
  out_bufs = (out_buf0, out_buf1)

  lane = lax.iota(jnp.int32, _L)
  idx_by_g = []
  for g in range(_GROUPS):
    t3 = _STRIDE * lane + (_STRIDE * _L) * g
    idx_by_g.append(_TILE * (t3 >> 7) + (t3 & 127))

  def start_in(g, slot):
    return pltpu.async_copy(
        x_hbm.at[pl.ds(in_base + g * _CHUNK_IN, _CHUNK_IN)],
        in_bufs[slot],
        in_sems.at[slot],
    )

  def start_out(g, slot):
    return pltpu.async_copy(
        out_bufs[slot],
        out_hbm.at[pl.ds(out_base + g * _CHUNK_OUT, _CHUNK_OUT)],
        out_sems.at[slot],
    )

  def wait_in(slot):
    pltpu.make_async_copy(
        x_hbm.at[pl.ds(in_base, _CHUNK_IN)], in_bufs[slot],
        in_sems.at[slot]).wait()

  def wait_out(slot):
    pltpu.make_async_copy(
        out_bufs[slot], out_hbm.at[pl.ds(out_base, _CHUNK_OUT)],
        out_sems.at[slot]).wait()

  def chunk_body(g, islot, oslot, first_out_use, start_next):
    wait_in(islot)
    if not first_out_use:
      wait_out(oslot)
    if start_next:
      start_in(g + 3, (islot + 3) % 4)
    _compact_chunk(in_bufs[islot], out_bufs[oslot], idx_by_g)
    start_out(g, oslot)

  for g in range(3):
    start_in(g, g)
  for g in (0, 1):
    chunk_body(g, g % 4, g % 2, True, True)

  steady_hi = _STEPS - 6
  def ring_body(h, _):
    g0 = 4 * h + 2
    for j in range(4):
      chunk_body(g0 + j, (2 + j) % 4, j % 2, False, True)
    return 0

  lax.fori_loop(0, (steady_hi - 2) // 4, ring_body, 0, unroll=False)

  for g in range(steady_hi, _STEPS):
    chunk_body(g, g % 4, g % 2, False, g + 3 < _STEPS)
  wait_out(0)
  wait_out(1)


@jax.jit
def kernel(x):
  x_tiles = x.reshape(_B, _CB, _SUB, _JI, _LANES).transpose(0, 1, 3, 2, 4)
  x_flat = x_tiles.reshape(-1)
  mesh = plsc.VectorSubcoreMesh(core_axis_name="c", subcore_axis_name="s")
  out_flat = pl.kernel(
      _sc_body,
      out_type=jax.ShapeDtypeStruct((_OUT_TOTAL,), jnp.float32),
      mesh=mesh,
      scratch_types=[
          pltpu.VMEM((_CHUNK_IN,), jnp.float32),
          pltpu.VMEM((_CHUNK_IN,), jnp.float32),
          pltpu.VMEM((_CHUNK_IN,), jnp.float32),
          pltpu.VMEM((_CHUNK_IN,), jnp.float32),
          pltpu.VMEM((_CHUNK_OUT,), jnp.float32),
          pltpu.VMEM((_CHUNK_OUT,), jnp.float32),
          pltpu.SemaphoreType.DMA((4,)),
          pltpu.SemaphoreType.DMA((2,)),
      ],
      compiler_params=pltpu.CompilerParams(needs_layout_passes=False),
      name="resample_nearest_sc",
  )(x_flat)
  out_tiles = out_flat.reshape(_B, _CB, _JO, _SUB, _LANES)
  return out_tiles.transpose(0, 1, 3, 2, 4).reshape(_B, _C, _T_OUT)
